# Initial kernel scaffold; baseline (speedup 1.0000x reference)
import jax
import jax.numpy as jnp
from jax.experimental import pallas as pl


def kernel(x, edge_index, W1, att_src1, att_dst1, b1, W2, att_src2, att_dst2, b2):
    raise NotImplementedError("write your pallas kernel here")


# speedup vs baseline: 30.4621x; 30.4621x over previous
import functools

import jax
import jax.numpy as jnp
from jax import lax
from jax.experimental import pallas as pl
from jax.experimental.pallas import tpu as pltpu
from jax.experimental.pallas import tpu_sc as plsc

N = 10000
E = 320000
IN = 128
HID = 8
HEADS = 8
F = HEADS * HID

NC, NS, L = 2, 16, 16
NW = NC * NS
EPT = E // NW
K = 80
NCHUNK = EPT // K
GP = K // L
NP = 10240
ROWW = 128
ROWS_PER_TILE = NP // NS
NEG = 0.2

_mesh = plsc.VectorSubcoreMesh(core_axis_name="c", subcore_axis_name="s")


def _sc1_body(esrc, edst, tA, m1, accO,
              sbuf, dbuf, arows, drows, mbuf, m1b, acc_sh, sem):
    c = lax.axis_index("c")
    s = lax.axis_index("s")
    w = c * NS + s

    if True:
        zero = jnp.zeros((L,), jnp.float32)

        def zrow(r, carry):
            for j in range(ROWW // L):
                mbuf[r, pl.ds(j * L, L)] = zero
            return carry
        lax.fori_loop(0, K, zrow, 0)

        base0 = s * ROWS_PER_TILE
        for j in range(8):
            pltpu.sync_copy(mbuf, acc_sh.at[pl.ds(base0 + j * 80, 80), :])
        pltpu.sync_copy(m1, m1b)
        plsc.subcore_barrier()

        iot = lax.iota(jnp.int32, L)

        def col(v):
            return jnp.full((L,), v, jnp.int32)

        def chunk(i, carry):
            base = w * EPT + i * K
            pltpu.sync_copy(esrc.at[pl.ds(base, K)], sbuf)
            pltpu.sync_copy(edst.at[pl.ds(base, K)], dbuf)
            cp1 = pltpu.async_copy(tA.at[sbuf], arows, sem)
            cp2 = pltpu.async_copy(tA.at[dbuf], drows, sem)
            cp1.wait()
            cp2.wait()
            for g in range(GP):
                erow = iot + (g * L)
                for h in range(HEADS):
                    a_s = plsc.load_gather(arows, [erow, col(64 + h)])
                    a_d = plsc.load_gather(drows, [erow, col(72 + h)])
                    a = a_s + a_d
                    a = jnp.where(a >= 0.0, a, a * NEG)
                    ex = jnp.exp(a - m1b[h, :])
                    plsc.store_scatter(mbuf, [erow, col(64 + h)], ex)
                    for cc in range(HID):
                        j = h * HID + cc
                        hv = plsc.load_gather(arows, [erow, col(j)])
                        plsc.store_scatter(mbuf, [erow, col(j)], hv * ex)
            pltpu.sync_copy(mbuf, acc_sh.at[dbuf], add=True)
            return carry
        lax.fori_loop(0, NCHUNK, chunk, 0)

        plsc.subcore_barrier()
        for j in range(8):
            pltpu.sync_copy(acc_sh.at[pl.ds(base0 + j * 80, 80), :], mbuf)
            pltpu.sync_copy(mbuf, accO.at[c, pl.ds(base0 + j * 80, 80), :])


_sc_params = pltpu.CompilerParams(needs_layout_passes=False)

_sc1 = functools.partial(
    pl.kernel,
    out_type=jax.ShapeDtypeStruct((NC, NP, ROWW), jnp.float32),
    mesh=_mesh,
    compiler_params=_sc_params,
    scratch_types=[
        pltpu.VMEM((K,), jnp.int32),
        pltpu.VMEM((K,), jnp.int32),
        pltpu.VMEM((K, ROWW), jnp.float32),
        pltpu.VMEM((K, ROWW), jnp.float32),
        pltpu.VMEM((K, ROWW), jnp.float32),
        pltpu.VMEM((8, 16), jnp.float32),
        pltpu.VMEM_SHARED((NP, ROWW), jnp.float32),
        pltpu.SemaphoreType.DMA,
    ],
)(_sc1_body)


def _sc2_body(esrc, edst, t2, m2, accO,
              sbuf, dbuf, srows, drows, mbuf, m2b, acc_sh, sem):
    c = lax.axis_index("c")
    s = lax.axis_index("s")
    w = c * NS + s

    if True:
        zero = jnp.zeros((L,), jnp.float32)

        def zrow(r, carry):
            for j in range(ROWW // L):
                mbuf[r, pl.ds(j * L, L)] = zero
            return carry
        lax.fori_loop(0, K, zrow, 0)

        base0 = s * ROWS_PER_TILE
        for j in range(8):
            pltpu.sync_copy(mbuf, acc_sh.at[pl.ds(base0 + j * 80, 80), :])
        pltpu.sync_copy(m2, m2b)
        plsc.subcore_barrier()

        iot = lax.iota(jnp.int32, L)

        def col(v):
            return jnp.full((L,), v, jnp.int32)

        def chunk(i, carry):
            base = w * EPT + i * K
            pltpu.sync_copy(esrc.at[pl.ds(base, K)], sbuf)
            pltpu.sync_copy(edst.at[pl.ds(base, K)], dbuf)
            cp1 = pltpu.async_copy(t2.at[sbuf], srows, sem)
            cp2 = pltpu.async_copy(t2.at[dbuf], drows, sem)
            cp1.wait()
            cp2.wait()
            m2v = m2b[0, :]
            for g in range(GP):
                erow = iot + (g * L)
                a_s = plsc.load_gather(srows, [erow, col(0)])
                a_d = plsc.load_gather(drows, [erow, col(1)])
                h2 = plsc.load_gather(srows, [erow, col(2)])
                a = a_s + a_d
                a = jnp.where(a >= 0.0, a, a * NEG)
                ex = jnp.exp(a - m2v)
                plsc.store_scatter(mbuf, [erow, col(0)], ex * h2)
                plsc.store_scatter(mbuf, [erow, col(1)], ex)
            pltpu.sync_copy(mbuf, acc_sh.at[dbuf], add=True)
            return carry
        lax.fori_loop(0, NCHUNK, chunk, 0)

        plsc.subcore_barrier()
        for j in range(8):
            pltpu.sync_copy(acc_sh.at[pl.ds(base0 + j * 80, 80), :], mbuf)
            pltpu.sync_copy(mbuf, accO.at[c, pl.ds(base0 + j * 80, 80), :])


_sc2 = functools.partial(
    pl.kernel,
    out_type=jax.ShapeDtypeStruct((NC, NP, ROWW), jnp.float32),
    mesh=_mesh,
    compiler_params=_sc_params,
    scratch_types=[
        pltpu.VMEM((K,), jnp.int32),
        pltpu.VMEM((K,), jnp.int32),
        pltpu.VMEM((K, ROWW), jnp.float32),
        pltpu.VMEM((K, ROWW), jnp.float32),
        pltpu.VMEM((K, ROWW), jnp.float32),
        pltpu.VMEM((1, 16), jnp.float32),
        pltpu.VMEM_SHARED((NP, ROWW), jnp.float32),
        pltpu.SemaphoreType.DMA,
    ],
)(_sc2_body)


B1 = 1000


def _tcA_body(x_ref, w1t_ref, asd_ref, tA_ref, mx_ref):
    h = jnp.dot(x_ref[...], w1t_ref[...], preferred_element_type=jnp.float32)
    asd = jnp.dot(h, asd_ref[...], preferred_element_type=jnp.float32)
    tA_ref[:, 0:F] = h
    tA_ref[:, F:80] = asd
    tA_ref[:, 80:ROWW] = jnp.zeros((B1, ROWW - 80), jnp.float32)
    cur = jnp.broadcast_to(jnp.max(asd, axis=0, keepdims=True), (8, 16))

    @pl.when(pl.program_id(0) == 0)
    def _():
        mx_ref[...] = cur

    @pl.when(pl.program_id(0) != 0)
    def _():
        mx_ref[...] = jnp.maximum(mx_ref[...], cur)


def _tcA(x, w1t, asd):
    return pl.pallas_call(
        _tcA_body,
        grid=(N // B1,),
        in_specs=[
            pl.BlockSpec((B1, IN), lambda i: (i, 0)),
            pl.BlockSpec((IN, F), lambda i: (0, 0)),
            pl.BlockSpec((F, 16), lambda i: (0, 0)),
        ],
        out_specs=[
            pl.BlockSpec((B1, ROWW), lambda i: (i, 0)),
            pl.BlockSpec((8, 16), lambda i: (0, 0)),
        ],
        out_shape=[
            jax.ShapeDtypeStruct((N, ROWW), jnp.float32),
            jax.ShapeDtypeStruct((8, 16), jnp.float32),
        ],
    )(x, w1t, asd)


def _tcB_body(acc_ref, tA_ref, m1_ref, e8_ref, b1_ref,
              w2r_ref, w2s_ref, w2d_ref, t2_ref, mx2_ref):
    a0 = acc_ref[0, :, :]
    a1 = acc_ref[1, :, :]
    hsrc = tA_ref[:, 0:F]
    a_s = tA_ref[:, F:F + HEADS]
    a_d = tA_ref[:, F + HEADS:80]
    del_unused = 0
    z = a_s + a_d
    z = jnp.where(z >= 0.0, z, z * NEG)
    ex8 = jnp.exp(z - m1_ref[...])
    e8 = e8_ref[...]
    num = a0[:, 0:F] + a1[:, 0:F] + hsrc * jnp.dot(
        ex8, e8, preferred_element_type=jnp.float32)
    den8 = a0[:, F:F + HEADS] + a1[:, F:F + HEADS] + ex8
    den = jnp.dot(den8, e8, preferred_element_type=jnp.float32) + 1e-16
    h1 = num / den + b1_ref[...]
    h1 = jnp.where(h1 > 0.0, h1, jnp.exp(h1) - 1.0)
    h2 = jnp.sum(h1 * w2r_ref[...], axis=1, keepdims=True)
    a2s = jnp.sum(h1 * w2s_ref[...], axis=1, keepdims=True)
    a2d = jnp.sum(h1 * w2d_ref[...], axis=1, keepdims=True)
    t2 = jnp.concatenate(
        [a2s, a2d, h2, jnp.zeros((B1, ROWW - 3), jnp.float32)], axis=1)
    t2_ref[...] = t2
    cur = jnp.broadcast_to(
        jnp.max(t2[:, 0:16], axis=0, keepdims=True), (8, 16))

    @pl.when(pl.program_id(0) == 0)
    def _():
        mx2_ref[...] = cur

    @pl.when(pl.program_id(0) != 0)
    def _():
        mx2_ref[...] = jnp.maximum(mx2_ref[...], cur)


def _tcB(acc, tA, m1row, e8, b1row, w2r, w2s, w2d):
    return pl.pallas_call(
        _tcB_body,
        grid=(N // B1,),
        in_specs=[
            pl.BlockSpec((NC, B1, ROWW), lambda i: (0, i, 0)),
            pl.BlockSpec((B1, ROWW), lambda i: (i, 0)),
            pl.BlockSpec((1, HEADS), lambda i: (0, 0)),
            pl.BlockSpec((HEADS, F), lambda i: (0, 0)),
            pl.BlockSpec((1, F), lambda i: (0, 0)),
            pl.BlockSpec((1, F), lambda i: (0, 0)),
            pl.BlockSpec((1, F), lambda i: (0, 0)),
            pl.BlockSpec((1, F), lambda i: (0, 0)),
        ],
        out_specs=[
            pl.BlockSpec((B1, ROWW), lambda i: (i, 0)),
            pl.BlockSpec((8, 16), lambda i: (0, 0)),
        ],
        out_shape=[
            jax.ShapeDtypeStruct((N, ROWW), jnp.float32),
            jax.ShapeDtypeStruct((8, 16), jnp.float32),
        ],
    )(acc, tA, m1row, e8, b1row, w2r, w2s, w2d)


def _tcC_body(acc_ref, t2_ref, m2_ref, out_ref):
    a0 = acc_ref[0, :, :]
    a1 = acc_ref[1, :, :]
    a2s = t2_ref[:, 0:1]
    a2d = t2_ref[:, 1:2]
    h2 = t2_ref[:, 2:3]
    z = a2s + a2d
    z = jnp.where(z >= 0.0, z, z * NEG)
    ex = jnp.exp(z - m2_ref[0:1, 0:1])
    num = a0[:, 0:1] + a1[:, 0:1] + ex * h2
    den = a0[:, 1:2] + a1[:, 1:2] + ex + 1e-16
    out_ref[...] = jnp.broadcast_to(num / den, (B1, 8))


def _tcC(acc2, t2, m2row):
    return pl.pallas_call(
        _tcC_body,
        grid=(N // B1,),
        in_specs=[
            pl.BlockSpec((NC, B1, ROWW), lambda i: (0, i, 0)),
            pl.BlockSpec((B1, ROWW), lambda i: (i, 0)),
            pl.BlockSpec((1, 16), lambda i: (0, 0)),
        ],
        out_specs=pl.BlockSpec((B1, 8), lambda i: (i, 0)),
        out_shape=jax.ShapeDtypeStruct((N, 8), jnp.float32),
    )(acc2, t2, m2row)


def kernel(x, edge_index, W1, att_src1, att_dst1, b1,
           W2, att_src2, att_dst2, b2):
    w1t = W1.T
    asrc = att_src1.reshape(F)
    adst = att_dst1.reshape(F)
    j = jnp.arange(F)[:, None] // HID
    hd = jnp.arange(HEADS)[None, :]
    As = jnp.where(j == hd, asrc[:, None], 0.0)
    Ad = jnp.where(j == hd, adst[:, None], 0.0)
    asd = jnp.concatenate([As, Ad], axis=1)
    e8 = jnp.kron(jnp.eye(HEADS, dtype=jnp.float32),
                  jnp.ones((1, HID), jnp.float32))
    b1row = b1.reshape(1, F)
    w2r = W2.reshape(1, F)
    w2s = w2r * att_src2.reshape(())
    w2d = w2r * att_dst2.reshape(())

    tA, mx = _tcA(x, w1t, asd)
    zmax = mx[0, 0:HEADS] + mx[0, HEADS:16]
    m1 = jnp.where(zmax >= 0.0, zmax, zmax * NEG)
    m1cols = jnp.broadcast_to(m1[:, None], (8, 16)) * jnp.ones((8, 16))
    m1row = m1.reshape(1, HEADS)

    esrc = edge_index[0]
    edst = edge_index[1]
    acc1 = _sc1(esrc, edst, tA, m1cols)

    t2, mx2 = _tcB(acc1, tA, m1row, e8, b1row, w2r, w2s, w2d)
    z2 = mx2[0, 0] + mx2[0, 1]
    m2 = jnp.where(z2 >= 0.0, z2, z2 * NEG)
    m2row = jnp.full((1, 16), m2, jnp.float32)

    acc2 = _sc2(esrc, edst, t2, m2row)

    out = _tcC(acc2, t2, m2row)
    return out[:, 0] + b2[0]


# speedup vs baseline: 32.7586x; 1.0754x over previous
import functools

import jax
import jax.numpy as jnp
from jax import lax
from jax.experimental import pallas as pl
from jax.experimental.pallas import tpu as pltpu
from jax.experimental.pallas import tpu_sc as plsc

N = 10000
E = 320000
IN = 128
HID = 8
HEADS = 8
F = HEADS * HID

NC, NS, L = 2, 16, 16
NW = NC * NS
EPT = E // NW
K = 80
NCHUNK = EPT // K
GP = K // L
NP = 10240
ROWW = 128
SROWS = 128
ROWS_PER_TILE = NP // NS
NEG = 0.2

_mesh = plsc.VectorSubcoreMesh(core_axis_name="c", subcore_axis_name="s")


def _sc1_body(esrc, edst, tA, m1, accO,
              sbuf, dbuf, arows, drows, mbuf, m1b, acc_sh, sem):
    c = lax.axis_index("c")
    s = lax.axis_index("s")
    w = c * NS + s

    if True:
        zero = jnp.zeros((L,), jnp.float32)

        def zrow(r, carry):
            for j in range(ROWW // L):
                mbuf[r, pl.ds(j * L, L)] = zero
            return carry
        lax.fori_loop(0, K, zrow, 0)

        base0 = s * ROWS_PER_TILE
        for j in range(8):
            pltpu.sync_copy(mbuf, acc_sh.at[pl.ds(base0 + j * 80, 80), :])
        pltpu.sync_copy(m1, m1b)
        plsc.subcore_barrier()

        iot = lax.iota(jnp.int32, L)

        def col(v):
            return jnp.full((L,), v, jnp.int32)

        def chunk(i, carry):
            base = w * EPT + i * K
            pltpu.sync_copy(esrc.at[pl.ds(base, K)], sbuf)
            pltpu.sync_copy(edst.at[pl.ds(base, K)], dbuf)
            cp1 = pltpu.async_copy(tA.at[sbuf], arows, sem)
            cp2 = pltpu.async_copy(tA.at[dbuf], drows, sem)
            cp1.wait()
            cp2.wait()
            for g in range(GP):
                erow = iot + (g * L)
                for h in range(HEADS):
                    a_s = plsc.load_gather(arows, [erow, col(64 + h)])
                    a_d = plsc.load_gather(drows, [erow, col(72 + h)])
                    a = a_s + a_d
                    a = jnp.where(a >= 0.0, a, a * NEG)
                    ex = jnp.exp(a - m1b[h, :])
                    plsc.store_scatter(mbuf, [erow, col(64 + h)], ex)
                    for cc in range(HID):
                        j = h * HID + cc
                        hv = plsc.load_gather(arows, [erow, col(j)])
                        plsc.store_scatter(mbuf, [erow, col(j)], hv * ex)
            pltpu.sync_copy(mbuf, acc_sh.at[dbuf], add=True)
            return carry
        lax.fori_loop(0, NCHUNK, chunk, 0)

        plsc.subcore_barrier()
        for j in range(8):
            pltpu.sync_copy(acc_sh.at[pl.ds(base0 + j * 80, 80), :], mbuf)
            pltpu.sync_copy(mbuf, accO.at[c, pl.ds(base0 + j * 80, 80), :])


_sc_params = pltpu.CompilerParams(needs_layout_passes=False)

_sc1 = functools.partial(
    pl.kernel,
    out_type=jax.ShapeDtypeStruct((NC, NP, ROWW), jnp.float32),
    mesh=_mesh,
    compiler_params=_sc_params,
    scratch_types=[
        pltpu.VMEM((K,), jnp.int32),
        pltpu.VMEM((K,), jnp.int32),
        pltpu.VMEM((K, ROWW), jnp.float32),
        pltpu.VMEM((K, ROWW), jnp.float32),
        pltpu.VMEM((K, ROWW), jnp.float32),
        pltpu.VMEM((8, 16), jnp.float32),
        pltpu.VMEM_SHARED((NP, ROWW), jnp.float32),
        pltpu.SemaphoreType.DMA,
    ],
)(_sc1_body)


def _sc2_body(esrc, edst, a2s, a2d, h2a, m2, accON, accOD,
              sbuf, dbuf, sgat, dgat, hgat, mnum, mden, stage, m2b,
              acc_n, acc_d, sem):
    c = lax.axis_index("c")
    s = lax.axis_index("s")
    w = c * NS + s

    if True:
        zero = jnp.zeros((L,), jnp.float32)

        def zrow(r, carry):
            stage[pl.ds(r * L, L)] = zero
            return carry
        lax.fori_loop(0, SROWS // L, zrow, 0)

        base0 = s * ROWS_PER_TILE
        for j in range(5):
            pltpu.sync_copy(stage, acc_n.at[pl.ds(base0 + j * SROWS, SROWS)])
            pltpu.sync_copy(stage, acc_d.at[pl.ds(base0 + j * SROWS, SROWS)])
        pltpu.sync_copy(m2, m2b)
        plsc.subcore_barrier()

        def chunk(i, carry):
            base = w * EPT + i * K
            pltpu.sync_copy(esrc.at[pl.ds(base, K)], sbuf)
            pltpu.sync_copy(edst.at[pl.ds(base, K)], dbuf)
            cp1 = pltpu.async_copy(a2s.at[sbuf], sgat, sem)
            cp2 = pltpu.async_copy(a2d.at[dbuf], dgat, sem)
            cp3 = pltpu.async_copy(h2a.at[sbuf], hgat, sem)
            cp1.wait()
            cp2.wait()
            cp3.wait()
            m2v = m2b[0, :]
            for g in range(GP):
                sl = pl.ds(g * L, L)
                a = sgat[sl] + dgat[sl]
                a = jnp.where(a >= 0.0, a, a * NEG)
                ex = jnp.exp(a - m2v)
                mnum[sl] = ex * hgat[sl]
                mden[sl] = ex
            pltpu.sync_copy(mnum, acc_n.at[dbuf], add=True)
            pltpu.sync_copy(mden, acc_d.at[dbuf], add=True)
            return carry
        lax.fori_loop(0, NCHUNK, chunk, 0)

        plsc.subcore_barrier()
        for j in range(5):
            pltpu.sync_copy(acc_n.at[pl.ds(base0 + j * SROWS, SROWS)], stage)
            pltpu.sync_copy(stage, accON.at[c, 0, pl.ds(base0 + j * SROWS, SROWS)])
            pltpu.sync_copy(acc_d.at[pl.ds(base0 + j * SROWS, SROWS)], stage)
            pltpu.sync_copy(stage, accOD.at[c, 0, pl.ds(base0 + j * SROWS, SROWS)])


_sc2 = functools.partial(
    pl.kernel,
    out_type=[jax.ShapeDtypeStruct((NC, 1, NP), jnp.float32),
              jax.ShapeDtypeStruct((NC, 1, NP), jnp.float32)],
    mesh=_mesh,
    compiler_params=_sc_params,
    scratch_types=[
        pltpu.VMEM((K,), jnp.int32),
        pltpu.VMEM((K,), jnp.int32),
        pltpu.VMEM((K,), jnp.float32),
        pltpu.VMEM((K,), jnp.float32),
        pltpu.VMEM((K,), jnp.float32),
        pltpu.VMEM((K,), jnp.float32),
        pltpu.VMEM((K,), jnp.float32),
        pltpu.VMEM((SROWS,), jnp.float32),
        pltpu.VMEM((1, 16), jnp.float32),
        pltpu.VMEM_SHARED((NP,), jnp.float32),
        pltpu.VMEM_SHARED((NP,), jnp.float32),
        pltpu.SemaphoreType.DMA,
    ],
)(_sc2_body)


B1 = 1000


def _tcA_body(x_ref, w1t_ref, asd_ref, tA_ref, mx_ref):
    h = jnp.dot(x_ref[...], w1t_ref[...], preferred_element_type=jnp.float32)
    asd = jnp.dot(h, asd_ref[...], preferred_element_type=jnp.float32)
    tA_ref[:, 0:F] = h
    tA_ref[:, F:80] = asd
    tA_ref[:, 80:ROWW] = jnp.zeros((B1, ROWW - 80), jnp.float32)
    cur = jnp.broadcast_to(jnp.max(asd, axis=0, keepdims=True), (8, 16))

    @pl.when(pl.program_id(0) == 0)
    def _():
        mx_ref[...] = cur

    @pl.when(pl.program_id(0) != 0)
    def _():
        mx_ref[...] = jnp.maximum(mx_ref[...], cur)


def _tcA(x, w1t, asd):
    return pl.pallas_call(
        _tcA_body,
        grid=(N // B1,),
        in_specs=[
            pl.BlockSpec((B1, IN), lambda i: (i, 0)),
            pl.BlockSpec((IN, F), lambda i: (0, 0)),
            pl.BlockSpec((F, 16), lambda i: (0, 0)),
        ],
        out_specs=[
            pl.BlockSpec((B1, ROWW), lambda i: (i, 0)),
            pl.BlockSpec((8, 16), lambda i: (0, 0)),
        ],
        out_shape=[
            jax.ShapeDtypeStruct((N, ROWW), jnp.float32),
            jax.ShapeDtypeStruct((8, 16), jnp.float32),
        ],
    )(x, w1t, asd)


def _tcB_body(acc_ref, tA_ref, m1_ref, e8_ref, b1_ref,
              w2r_ref, w2s_ref, w2d_ref, t2_ref, mx2_ref):
    a0 = acc_ref[0, :, :]
    a1 = acc_ref[1, :, :]
    hsrc = tA_ref[:, 0:F]
    a_s = tA_ref[:, F:F + HEADS]
    a_d = tA_ref[:, F + HEADS:80]
    del_unused = 0
    z = a_s + a_d
    z = jnp.where(z >= 0.0, z, z * NEG)
    ex8 = jnp.exp(z - m1_ref[...])
    e8 = e8_ref[...]
    num = a0[:, 0:F] + a1[:, 0:F] + hsrc * jnp.dot(
        ex8, e8, preferred_element_type=jnp.float32)
    den8 = a0[:, F:F + HEADS] + a1[:, F:F + HEADS] + ex8
    den = jnp.dot(den8, e8, preferred_element_type=jnp.float32) + 1e-16
    h1 = num / den + b1_ref[...]
    h1 = jnp.where(h1 > 0.0, h1, jnp.exp(h1) - 1.0)
    h2 = jnp.sum(h1 * w2r_ref[...], axis=1, keepdims=True)
    a2s = jnp.sum(h1 * w2s_ref[...], axis=1, keepdims=True)
    a2d = jnp.sum(h1 * w2d_ref[...], axis=1, keepdims=True)
    t2 = jnp.concatenate(
        [a2s, a2d, h2, jnp.zeros((B1, ROWW - 3), jnp.float32)], axis=1)
    t2_ref[...] = t2
    cur = jnp.broadcast_to(
        jnp.max(t2[:, 0:16], axis=0, keepdims=True), (8, 16))

    @pl.when(pl.program_id(0) == 0)
    def _():
        mx2_ref[...] = cur

    @pl.when(pl.program_id(0) != 0)
    def _():
        mx2_ref[...] = jnp.maximum(mx2_ref[...], cur)


def _tcB(acc, tA, m1row, e8, b1row, w2r, w2s, w2d):
    return pl.pallas_call(
        _tcB_body,
        grid=(N // B1,),
        in_specs=[
            pl.BlockSpec((NC, B1, ROWW), lambda i: (0, i, 0)),
            pl.BlockSpec((B1, ROWW), lambda i: (i, 0)),
            pl.BlockSpec((1, HEADS), lambda i: (0, 0)),
            pl.BlockSpec((HEADS, F), lambda i: (0, 0)),
            pl.BlockSpec((1, F), lambda i: (0, 0)),
            pl.BlockSpec((1, F), lambda i: (0, 0)),
            pl.BlockSpec((1, F), lambda i: (0, 0)),
            pl.BlockSpec((1, F), lambda i: (0, 0)),
        ],
        out_specs=[
            pl.BlockSpec((B1, ROWW), lambda i: (i, 0)),
            pl.BlockSpec((8, 16), lambda i: (0, 0)),
        ],
        out_shape=[
            jax.ShapeDtypeStruct((N, ROWW), jnp.float32),
            jax.ShapeDtypeStruct((8, 16), jnp.float32),
        ],
    )(acc, tA, m1row, e8, b1row, w2r, w2s, w2d)


def _tcC_body(acc_ref, t2_ref, m2_ref, out_ref):
    a2s = t2_ref[:, 0:1]
    a2d = t2_ref[:, 1:2]
    h2 = t2_ref[:, 2:3]
    z = a2s + a2d
    z = jnp.where(z >= 0.0, z, z * NEG)
    ex = jnp.exp(z - m2_ref[0:1, 0:1])
    num = acc_ref[:, 0:1] + acc_ref[:, 1:2] + ex * h2
    den = acc_ref[:, 2:3] + acc_ref[:, 3:4] + ex + 1e-16
    out_ref[...] = jnp.broadcast_to(num / den, (B1, 8))


def _tcC(accc, t2, m2row):
    return pl.pallas_call(
        _tcC_body,
        grid=(N // B1,),
        in_specs=[
            pl.BlockSpec((B1, 4), lambda i: (i, 0)),
            pl.BlockSpec((B1, ROWW), lambda i: (i, 0)),
            pl.BlockSpec((1, 16), lambda i: (0, 0)),
        ],
        out_specs=pl.BlockSpec((B1, 8), lambda i: (i, 0)),
        out_shape=jax.ShapeDtypeStruct((N, 8), jnp.float32),
    )(accc, t2, m2row)


def kernel(x, edge_index, W1, att_src1, att_dst1, b1,
           W2, att_src2, att_dst2, b2):
    w1t = W1.T
    asrc = att_src1.reshape(F)
    adst = att_dst1.reshape(F)
    j = jnp.arange(F)[:, None] // HID
    hd = jnp.arange(HEADS)[None, :]
    As = jnp.where(j == hd, asrc[:, None], 0.0)
    Ad = jnp.where(j == hd, adst[:, None], 0.0)
    asd = jnp.concatenate([As, Ad], axis=1)
    e8 = jnp.kron(jnp.eye(HEADS, dtype=jnp.float32),
                  jnp.ones((1, HID), jnp.float32))
    b1row = b1.reshape(1, F)
    w2r = W2.reshape(1, F)
    w2s = w2r * att_src2.reshape(())
    w2d = w2r * att_dst2.reshape(())

    tA, mx = _tcA(x, w1t, asd)
    zmax = mx[0, 0:HEADS] + mx[0, HEADS:16]
    m1 = jnp.where(zmax >= 0.0, zmax, zmax * NEG)
    m1cols = jnp.broadcast_to(m1[:, None], (8, 16)) * jnp.ones((8, 16))
    m1row = m1.reshape(1, HEADS)

    esrc = edge_index[0]
    edst = edge_index[1]
    acc1 = _sc1(esrc, edst, tA, m1cols)

    t2, mx2 = _tcB(acc1, tA, m1row, e8, b1row, w2r, w2s, w2d)
    z2 = mx2[0, 0] + mx2[0, 1]
    m2 = jnp.where(z2 >= 0.0, z2, z2 * NEG)
    m2row = jnp.full((1, 16), m2, jnp.float32)

    a2s1 = t2[:, 0]
    a2d1 = t2[:, 1]
    h2a1 = t2[:, 2]
    acc2n, acc2d = _sc2(esrc, edst, a2s1, a2d1, h2a1, m2row)

    accc = jnp.stack(
        [acc2n[0, 0], acc2n[1, 0], acc2d[0, 0], acc2d[1, 0]], axis=-1)
    out = _tcC(accc[:N], t2, m2row)
    return out[:, 0] + b2[0]


# speedup vs baseline: 36.3611x; 1.1100x over previous
import functools

import jax
import jax.numpy as jnp
from jax import lax
from jax.experimental import pallas as pl
from jax.experimental.pallas import tpu as pltpu
from jax.experimental.pallas import tpu_sc as plsc

N = 10000
E = 320000
IN = 128
HID = 8
HEADS = 8
F = HEADS * HID

NC, NS, L = 2, 16, 16
NW = NC * NS
EPT = E // NW
K = 80
NCHUNK = EPT // K
GP = K // L
NP = 10240
ROWW = 128
SROWS = 128
ROWS_PER_TILE = NP // NS
NEG = 0.2

_mesh = plsc.VectorSubcoreMesh(core_axis_name="c", subcore_axis_name="s")


def _sc1_body(esrc, edst, tA, m1, accO,
              sbuf, dbuf, arows, drows, mbuf, m1b, acc_sh, sem):
    c = lax.axis_index("c")
    s = lax.axis_index("s")
    w = c * NS + s

    zero = jnp.zeros((L,), jnp.float32)

    def zrow(r, carry):
        for j in range(ROWW // L):
            mbuf[r, pl.ds(j * L, L)] = zero
        return carry
    lax.fori_loop(0, K, zrow, 0)

    base0 = s * ROWS_PER_TILE
    for j in range(8):
        pltpu.sync_copy(mbuf, acc_sh.at[pl.ds(base0 + j * 80, 80), :])
    pltpu.sync_copy(m1, m1b)
    plsc.subcore_barrier()

    iot = lax.iota(jnp.int32, L)

    def col(v):
        return jnp.full((L,), v, jnp.int32)

    def chunk(i, carry):
        base = w * EPT + i * K
        pltpu.sync_copy(esrc.at[pl.ds(base, K)], sbuf)
        pltpu.sync_copy(edst.at[pl.ds(base, K)], dbuf)
        cp1 = pltpu.async_copy(tA.at[sbuf], arows, sem)
        cp2 = pltpu.async_copy(tA.at[dbuf], drows, sem)
        cp1.wait()
        cp2.wait()
        for g in range(GP):
            erow = iot + (g * L)
            for h in range(HEADS):
                a_s = plsc.load_gather(arows, [erow, col(64 + h)])
                a_d = plsc.load_gather(drows, [erow, col(72 + h)])
                a = a_s + a_d
                a = jnp.where(a >= 0.0, a, a * NEG)
                ex = jnp.exp(a - m1b[h, :])
                plsc.store_scatter(mbuf, [erow, col(64 + h)], ex)
                for cc in range(HID):
                    j = h * HID + cc
                    hv = plsc.load_gather(arows, [erow, col(j)])
                    plsc.store_scatter(mbuf, [erow, col(j)], hv * ex)
        pltpu.sync_copy(mbuf, acc_sh.at[dbuf], add=True)
        return carry
    lax.fori_loop(0, NCHUNK, chunk, 0)

    plsc.subcore_barrier()
    for j in range(8):
        pltpu.sync_copy(acc_sh.at[pl.ds(base0 + j * 80, 80), :], mbuf)
        pltpu.sync_copy(mbuf, accO.at[c, pl.ds(base0 + j * 80, 80), :])


_sc_params = pltpu.CompilerParams(needs_layout_passes=False)

_sc1 = functools.partial(
    pl.kernel,
    out_type=jax.ShapeDtypeStruct((NC, NP, ROWW), jnp.float32),
    mesh=_mesh,
    compiler_params=_sc_params,
    scratch_types=[
        pltpu.VMEM((K,), jnp.int32),
        pltpu.VMEM((K,), jnp.int32),
        pltpu.VMEM((K, ROWW), jnp.float32),
        pltpu.VMEM((K, ROWW), jnp.float32),
        pltpu.VMEM((K, ROWW), jnp.float32),
        pltpu.VMEM((8, 16), jnp.float32),
        pltpu.VMEM_SHARED((NP, ROWW), jnp.float32),
        pltpu.SemaphoreType.DMA,
    ],
)(_sc1_body)


def _sc2_body(esrc, edst, a2s, a2d, h2a, m2, accON, accOD,
              sidx, didx, dscat, six0, six1, dix0, dix1,
              sg0, sg1, dg0, dg1, hg0, hg1,
              mnum, mden, stage, m2b, acc_n, acc_d, sem0, sem1):
    c = lax.axis_index("c")
    s = lax.axis_index("s")
    w = c * NS + s
    sg = (sg0, sg1)
    dg = (dg0, dg1)
    hg = (hg0, hg1)
    six = (six0, six1)
    dix = (dix0, dix1)
    sems = (sem0, sem1)
    zero = jnp.zeros((L,), jnp.float32)

    def zrow(r, carry):
        stage[pl.ds(r * L, L)] = zero
        return carry
    lax.fori_loop(0, SROWS // L, zrow, 0)

    base0 = s * ROWS_PER_TILE
    for j in range(5):
        pltpu.sync_copy(stage, acc_n.at[pl.ds(base0 + j * SROWS, SROWS)])
        pltpu.sync_copy(stage, acc_d.at[pl.ds(base0 + j * SROWS, SROWS)])
    pltpu.sync_copy(m2, m2b)
    pltpu.sync_copy(esrc.at[pl.ds(w * EPT, EPT)], sidx)
    pltpu.sync_copy(edst.at[pl.ds(w * EPT, EPT)], didx)
    plsc.subcore_barrier()

    def issue(ic, b):
        for g in range(GP):
            gsl = pl.ds(g * L, L)
            six[b][gsl] = sidx[pl.ds(ic * K + g * L, L)]
            dix[b][gsl] = didx[pl.ds(ic * K + g * L, L)]
        pltpu.async_copy(a2s.at[six[b]], sg[b], sems[b])
        pltpu.async_copy(a2d.at[dix[b]], dg[b], sems[b])
        pltpu.async_copy(h2a.at[six[b]], hg[b], sems[b])

    def wait_gathers(b):
        pltpu.make_async_copy(a2s.at[six[b]], sg[b], sems[b]).wait()
        pltpu.make_async_copy(a2d.at[dix[b]], dg[b], sems[b]).wait()
        pltpu.make_async_copy(h2a.at[six[b]], hg[b], sems[b]).wait()

    def process(ic, b):
        m2v = m2b[0, :]
        for g in range(GP):
            sl = pl.ds(g * L, L)
            a = sg[b][sl] + dg[b][sl]
            a = jnp.where(a >= 0.0, a, a * NEG)
            ex = jnp.exp(a - m2v)
            mnum[sl] = ex * hg[b][sl]
            mden[sl] = ex
        for g in range(GP):
            dscat[pl.ds(g * L, L)] = didx[pl.ds(ic * K + g * L, L)]
        pltpu.sync_copy(mnum, acc_n.at[dscat], add=True)
        pltpu.sync_copy(mden, acc_d.at[dscat], add=True)

    issue(0, 0)
    issue(1, 1)

    def dbody(i, carry):
        ic0 = i * 2
        for b in range(2):
            ic = ic0 + b
            wait_gathers(b)
            process(ic, b)
            nxt = ic + 2

            @pl.when(nxt < NCHUNK)
            def _():
                issue(nxt, b)
        return carry
    lax.fori_loop(0, NCHUNK // 2, dbody, 0)

    wait_gathers(0)
    process(NCHUNK - 1, 0)

    plsc.subcore_barrier()
    for j in range(5):
        pltpu.sync_copy(acc_n.at[pl.ds(base0 + j * SROWS, SROWS)], stage)
        pltpu.sync_copy(stage, accON.at[c, 0, pl.ds(base0 + j * SROWS, SROWS)])
        pltpu.sync_copy(acc_d.at[pl.ds(base0 + j * SROWS, SROWS)], stage)
        pltpu.sync_copy(stage, accOD.at[c, 0, pl.ds(base0 + j * SROWS, SROWS)])


_sc2 = functools.partial(
    pl.kernel,
    out_type=[jax.ShapeDtypeStruct((NC, 1, NP), jnp.float32),
              jax.ShapeDtypeStruct((NC, 1, NP), jnp.float32)],
    mesh=_mesh,
    compiler_params=_sc_params,
    scratch_types=[
        pltpu.VMEM((EPT,), jnp.int32),
        pltpu.VMEM((EPT,), jnp.int32),
        pltpu.VMEM((K,), jnp.int32),
        pltpu.VMEM((K,), jnp.int32),
        pltpu.VMEM((K,), jnp.int32),
        pltpu.VMEM((K,), jnp.int32),
        pltpu.VMEM((K,), jnp.int32),
        pltpu.VMEM((K,), jnp.float32),
        pltpu.VMEM((K,), jnp.float32),
        pltpu.VMEM((K,), jnp.float32),
        pltpu.VMEM((K,), jnp.float32),
        pltpu.VMEM((K,), jnp.float32),
        pltpu.VMEM((K,), jnp.float32),
        pltpu.VMEM((K,), jnp.float32),
        pltpu.VMEM((K,), jnp.float32),
        pltpu.VMEM((SROWS,), jnp.float32),
        pltpu.VMEM((1, 16), jnp.float32),
        pltpu.VMEM_SHARED((NP,), jnp.float32),
        pltpu.VMEM_SHARED((NP,), jnp.float32),
        pltpu.SemaphoreType.DMA,
        pltpu.SemaphoreType.DMA,
    ],
)(_sc2_body)


B1 = 1000


def _tcA_body(x_ref, w1t_ref, asd_ref, tA_ref, mx_ref):
    h = jnp.dot(x_ref[...], w1t_ref[...], preferred_element_type=jnp.float32)
    asd = jnp.dot(h, asd_ref[...], preferred_element_type=jnp.float32)
    tA_ref[:, 0:F] = h
    tA_ref[:, F:80] = asd
    tA_ref[:, 80:ROWW] = jnp.zeros((B1, ROWW - 80), jnp.float32)
    cur = jnp.broadcast_to(jnp.max(asd, axis=0, keepdims=True), (8, 16))

    @pl.when(pl.program_id(0) == 0)
    def _():
        mx_ref[...] = cur

    @pl.when(pl.program_id(0) != 0)
    def _():
        mx_ref[...] = jnp.maximum(mx_ref[...], cur)


def _tcA(x, w1t, asd):
    return pl.pallas_call(
        _tcA_body,
        grid=(N // B1,),
        in_specs=[
            pl.BlockSpec((B1, IN), lambda i: (i, 0)),
            pl.BlockSpec((IN, F), lambda i: (0, 0)),
            pl.BlockSpec((F, 16), lambda i: (0, 0)),
        ],
        out_specs=[
            pl.BlockSpec((B1, ROWW), lambda i: (i, 0)),
            pl.BlockSpec((8, 16), lambda i: (0, 0)),
        ],
        out_shape=[
            jax.ShapeDtypeStruct((N, ROWW), jnp.float32),
            jax.ShapeDtypeStruct((8, 16), jnp.float32),
        ],
    )(x, w1t, asd)


def _tcB_body(acc_ref, tA_ref, m1_ref, e8_ref, b1_ref,
              w2r_ref, w2s_ref, w2d_ref, t2_ref, mx2_ref):
    a0 = acc_ref[0, :, :]
    a1 = acc_ref[1, :, :]
    hsrc = tA_ref[:, 0:F]
    a_s = tA_ref[:, F:F + HEADS]
    a_d = tA_ref[:, F + HEADS:80]
    del_unused = 0
    z = a_s + a_d
    z = jnp.where(z >= 0.0, z, z * NEG)
    ex8 = jnp.exp(z - m1_ref[...])
    e8 = e8_ref[...]
    num = a0[:, 0:F] + a1[:, 0:F] + hsrc * jnp.dot(
        ex8, e8, preferred_element_type=jnp.float32)
    den8 = a0[:, F:F + HEADS] + a1[:, F:F + HEADS] + ex8
    den = jnp.dot(den8, e8, preferred_element_type=jnp.float32) + 1e-16
    h1 = num / den + b1_ref[...]
    h1 = jnp.where(h1 > 0.0, h1, jnp.exp(h1) - 1.0)
    h2 = jnp.sum(h1 * w2r_ref[...], axis=1, keepdims=True)
    a2s = jnp.sum(h1 * w2s_ref[...], axis=1, keepdims=True)
    a2d = jnp.sum(h1 * w2d_ref[...], axis=1, keepdims=True)
    t2 = jnp.concatenate(
        [a2s, a2d, h2, jnp.zeros((B1, ROWW - 3), jnp.float32)], axis=1)
    t2_ref[...] = t2
    cur = jnp.broadcast_to(
        jnp.max(t2[:, 0:16], axis=0, keepdims=True), (8, 16))

    @pl.when(pl.program_id(0) == 0)
    def _():
        mx2_ref[...] = cur

    @pl.when(pl.program_id(0) != 0)
    def _():
        mx2_ref[...] = jnp.maximum(mx2_ref[...], cur)


def _tcB(acc, tA, m1row, e8, b1row, w2r, w2s, w2d):
    return pl.pallas_call(
        _tcB_body,
        grid=(N // B1,),
        in_specs=[
            pl.BlockSpec((NC, B1, ROWW), lambda i: (0, i, 0)),
            pl.BlockSpec((B1, ROWW), lambda i: (i, 0)),
            pl.BlockSpec((1, HEADS), lambda i: (0, 0)),
            pl.BlockSpec((HEADS, F), lambda i: (0, 0)),
            pl.BlockSpec((1, F), lambda i: (0, 0)),
            pl.BlockSpec((1, F), lambda i: (0, 0)),
            pl.BlockSpec((1, F), lambda i: (0, 0)),
            pl.BlockSpec((1, F), lambda i: (0, 0)),
        ],
        out_specs=[
            pl.BlockSpec((B1, ROWW), lambda i: (i, 0)),
            pl.BlockSpec((8, 16), lambda i: (0, 0)),
        ],
        out_shape=[
            jax.ShapeDtypeStruct((N, ROWW), jnp.float32),
            jax.ShapeDtypeStruct((8, 16), jnp.float32),
        ],
    )(acc, tA, m1row, e8, b1row, w2r, w2s, w2d)


def _tcC_body(acc_ref, t2_ref, m2_ref, out_ref):
    a2s = t2_ref[:, 0:1]
    a2d = t2_ref[:, 1:2]
    h2 = t2_ref[:, 2:3]
    z = a2s + a2d
    z = jnp.where(z >= 0.0, z, z * NEG)
    ex = jnp.exp(z - m2_ref[0:1, 0:1])
    num = acc_ref[:, 0:1] + acc_ref[:, 1:2] + ex * h2
    den = acc_ref[:, 2:3] + acc_ref[:, 3:4] + ex + 1e-16
    out_ref[...] = jnp.broadcast_to(num / den, (B1, 8))


def _tcC(accc, t2, m2row):
    return pl.pallas_call(
        _tcC_body,
        grid=(N // B1,),
        in_specs=[
            pl.BlockSpec((B1, 4), lambda i: (i, 0)),
            pl.BlockSpec((B1, ROWW), lambda i: (i, 0)),
            pl.BlockSpec((1, 16), lambda i: (0, 0)),
        ],
        out_specs=pl.BlockSpec((B1, 8), lambda i: (i, 0)),
        out_shape=jax.ShapeDtypeStruct((N, 8), jnp.float32),
    )(accc, t2, m2row)


def kernel(x, edge_index, W1, att_src1, att_dst1, b1,
           W2, att_src2, att_dst2, b2):
    w1t = W1.T
    asrc = att_src1.reshape(F)
    adst = att_dst1.reshape(F)
    j = jnp.arange(F)[:, None] // HID
    hd = jnp.arange(HEADS)[None, :]
    As = jnp.where(j == hd, asrc[:, None], 0.0)
    Ad = jnp.where(j == hd, adst[:, None], 0.0)
    asd = jnp.concatenate([As, Ad], axis=1)
    e8 = jnp.kron(jnp.eye(HEADS, dtype=jnp.float32),
                  jnp.ones((1, HID), jnp.float32))
    b1row = b1.reshape(1, F)
    w2r = W2.reshape(1, F)
    w2s = w2r * att_src2.reshape(())
    w2d = w2r * att_dst2.reshape(())

    tA, mx = _tcA(x, w1t, asd)
    zmax = mx[0, 0:HEADS] + mx[0, HEADS:16]
    m1 = jnp.where(zmax >= 0.0, zmax, zmax * NEG)
    m1cols = jnp.broadcast_to(m1[:, None], (8, 16)) * jnp.ones((8, 16))
    m1row = m1.reshape(1, HEADS)

    esrc = edge_index[0]
    edst = edge_index[1]
    acc1 = _sc1(esrc, edst, tA, m1cols)

    t2, mx2 = _tcB(acc1, tA, m1row, e8, b1row, w2r, w2s, w2d)
    z2 = mx2[0, 0] + mx2[0, 1]
    m2 = jnp.where(z2 >= 0.0, z2, z2 * NEG)
    m2row = jnp.full((1, 16), m2, jnp.float32)

    a2s1 = t2[:, 0]
    a2d1 = t2[:, 1]
    h2a1 = t2[:, 2]
    acc2n, acc2d = _sc2(esrc, edst, a2s1, a2d1, h2a1, m2row)

    accc = jnp.stack(
        [acc2n[0, 0], acc2n[1, 0], acc2d[0, 0], acc2d[1, 0]], axis=-1)
    out = _tcC(accc[:N], t2, m2row)
    return out[:, 0] + b2[0]


# speedup vs baseline: 39.9571x; 1.0989x over previous
import functools

import jax
import jax.numpy as jnp
from jax import lax
from jax.experimental import pallas as pl
from jax.experimental.pallas import tpu as pltpu
from jax.experimental.pallas import tpu_sc as plsc

N = 10000
E = 320000
IN = 128
HID = 8
HEADS = 8
F = HEADS * HID

NC, NS, L = 2, 16, 16
NW = NC * NS
EPT = E // NW
K = 80
NCHUNK = EPT // K
GP = K // L
NP = 10240
ROWW = 128
SROWS = 128
ROWS_PER_TILE = NP // NS
NEG = 0.2

_mesh = plsc.VectorSubcoreMesh(core_axis_name="c", subcore_axis_name="s")


def _sc1_body(esrc, edst, tA, m1, accO,
              sbufA, sbufB, dbufA, dbufB,
              arowsA, arowsB, drowsA, drowsB,
              m1b, acc_sh, semA, semB):
    c = lax.axis_index("c")
    s = lax.axis_index("s")
    w = c * NS + s

    zero = jnp.zeros((L,), jnp.float32)

    def zrow(r, carry):
        for j in range(ROWW // L):
            arowsA[r, pl.ds(j * L, L)] = zero
        return carry
    lax.fori_loop(0, K, zrow, 0)

    base0 = s * ROWS_PER_TILE
    for j in range(8):
        pltpu.sync_copy(arowsA, acc_sh.at[pl.ds(base0 + j * 80, 80), :])
    pltpu.sync_copy(m1, m1b)
    plsc.subcore_barrier()

    iot = lax.iota(jnp.int32, L)

    def col(v):
        return jnp.full((L,), v, jnp.int32)

    def fetch(base, sbuf, dbuf, ar, dr, sem):
        pltpu.sync_copy(esrc.at[pl.ds(base, K)], sbuf)
        pltpu.sync_copy(edst.at[pl.ds(base, K)], dbuf)
        return (pltpu.async_copy(tA.at[sbuf], ar, sem),
                pltpu.async_copy(tA.at[dbuf], dr, sem))

    def crunch(cps, dbuf, ar, dr):
        cps[0].wait()
        cps[1].wait()
        for g in range(GP):
            erow = iot + (g * L)
            for h in range(HEADS):
                a_s = plsc.load_gather(ar, [erow, col(64 + h)])
                a_d = plsc.load_gather(dr, [erow, col(72 + h)])
                a = a_s + a_d
                a = jnp.where(a >= 0.0, a, a * NEG)
                ex = jnp.exp(a - m1b[h, :])
                plsc.store_scatter(dr, [erow, col(64 + h)], ex)
                for cc in range(HID):
                    j = h * HID + cc
                    hv = plsc.load_gather(ar, [erow, col(j)])
                    plsc.store_scatter(dr, [erow, col(j)], hv * ex)
        pltpu.sync_copy(dr, acc_sh.at[dbuf], add=True)

    def pair(i, carry):
        baseA = w * EPT + i * (2 * K)
        cpsA = fetch(baseA, sbufA, dbufA, arowsA, drowsA, semA)
        cpsB = fetch(baseA + K, sbufB, dbufB, arowsB, drowsB, semB)
        crunch(cpsA, dbufA, arowsA, drowsA)
        crunch(cpsB, dbufB, arowsB, drowsB)
        return carry
    lax.fori_loop(0, NCHUNK // 2, pair, 0)

    cpsT = fetch(w * EPT + (NCHUNK - 1) * K, sbufA, dbufA,
                 arowsA, drowsA, semA)
    crunch(cpsT, dbufA, arowsA, drowsA)

    plsc.subcore_barrier()
    for j in range(8):
        pltpu.sync_copy(acc_sh.at[pl.ds(base0 + j * 80, 80), :], arowsA)
        pltpu.sync_copy(arowsA, accO.at[c, pl.ds(base0 + j * 80, 80), :])


_sc_params = pltpu.CompilerParams(needs_layout_passes=False)

_sc1 = functools.partial(
    pl.kernel,
    out_type=jax.ShapeDtypeStruct((NC, NP, ROWW), jnp.float32),
    mesh=_mesh,
    compiler_params=_sc_params,
    scratch_types=[
        pltpu.VMEM((K,), jnp.int32),
        pltpu.VMEM((K,), jnp.int32),
        pltpu.VMEM((K,), jnp.int32),
        pltpu.VMEM((K,), jnp.int32),
        pltpu.VMEM((K, ROWW), jnp.float32),
        pltpu.VMEM((K, ROWW), jnp.float32),
        pltpu.VMEM((K, ROWW), jnp.float32),
        pltpu.VMEM((K, ROWW), jnp.float32),
        pltpu.VMEM((8, 16), jnp.float32),
        pltpu.VMEM_SHARED((NP, ROWW), jnp.float32),
        pltpu.SemaphoreType.DMA,
        pltpu.SemaphoreType.DMA,
    ],
)(_sc1_body)


def _sc2_body(esrc, edst, a2s, a2d, h2a, m2, accON, accOD,
              sidx, didx, dscat, six0, six1, dix0, dix1,
              sg0, sg1, dg0, dg1, hg0, hg1,
              mnum, mden, stage, m2b, acc_n, acc_d, sem0, sem1):
    c = lax.axis_index("c")
    s = lax.axis_index("s")
    w = c * NS + s
    sg = (sg0, sg1)
    dg = (dg0, dg1)
    hg = (hg0, hg1)
    six = (six0, six1)
    dix = (dix0, dix1)
    sems = (sem0, sem1)
    zero = jnp.zeros((L,), jnp.float32)

    def zrow(r, carry):
        stage[pl.ds(r * L, L)] = zero
        return carry
    lax.fori_loop(0, SROWS // L, zrow, 0)

    base0 = s * ROWS_PER_TILE
    for j in range(5):
        pltpu.sync_copy(stage, acc_n.at[pl.ds(base0 + j * SROWS, SROWS)])
        pltpu.sync_copy(stage, acc_d.at[pl.ds(base0 + j * SROWS, SROWS)])
    pltpu.sync_copy(m2, m2b)
    pltpu.sync_copy(esrc.at[pl.ds(w * EPT, EPT)], sidx)
    pltpu.sync_copy(edst.at[pl.ds(w * EPT, EPT)], didx)
    plsc.subcore_barrier()

    def issue(ic, b):
        for g in range(GP):
            gsl = pl.ds(g * L, L)
            six[b][gsl] = sidx[pl.ds(ic * K + g * L, L)]
            dix[b][gsl] = didx[pl.ds(ic * K + g * L, L)]
        pltpu.async_copy(a2s.at[six[b]], sg[b], sems[b])
        pltpu.async_copy(a2d.at[dix[b]], dg[b], sems[b])
        pltpu.async_copy(h2a.at[six[b]], hg[b], sems[b])

    def wait_gathers(b):
        pltpu.make_async_copy(a2s.at[six[b]], sg[b], sems[b]).wait()
        pltpu.make_async_copy(a2d.at[dix[b]], dg[b], sems[b]).wait()
        pltpu.make_async_copy(h2a.at[six[b]], hg[b], sems[b]).wait()

    def process(ic, b):
        m2v = m2b[0, :]
        for g in range(GP):
            sl = pl.ds(g * L, L)
            a = sg[b][sl] + dg[b][sl]
            a = jnp.where(a >= 0.0, a, a * NEG)
            ex = jnp.exp(a - m2v)
            mnum[sl] = ex * hg[b][sl]
            mden[sl] = ex
        for g in range(GP):
            dscat[pl.ds(g * L, L)] = didx[pl.ds(ic * K + g * L, L)]
        pltpu.sync_copy(mnum, acc_n.at[dscat], add=True)
        pltpu.sync_copy(mden, acc_d.at[dscat], add=True)

    issue(0, 0)
    issue(1, 1)

    def dbody(i, carry):
        ic0 = i * 2
        for b in range(2):
            ic = ic0 + b
            wait_gathers(b)
            process(ic, b)
            nxt = ic + 2

            @pl.when(nxt < NCHUNK)
            def _():
                issue(nxt, b)
        return carry
    lax.fori_loop(0, NCHUNK // 2, dbody, 0)

    wait_gathers(0)
    process(NCHUNK - 1, 0)

    plsc.subcore_barrier()
    for j in range(5):
        pltpu.sync_copy(acc_n.at[pl.ds(base0 + j * SROWS, SROWS)], stage)
        pltpu.sync_copy(stage, accON.at[c, 0, pl.ds(base0 + j * SROWS, SROWS)])
        pltpu.sync_copy(acc_d.at[pl.ds(base0 + j * SROWS, SROWS)], stage)
        pltpu.sync_copy(stage, accOD.at[c, 0, pl.ds(base0 + j * SROWS, SROWS)])


_sc2 = functools.partial(
    pl.kernel,
    out_type=[jax.ShapeDtypeStruct((NC, 1, NP), jnp.float32),
              jax.ShapeDtypeStruct((NC, 1, NP), jnp.float32)],
    mesh=_mesh,
    compiler_params=_sc_params,
    scratch_types=[
        pltpu.VMEM((EPT,), jnp.int32),
        pltpu.VMEM((EPT,), jnp.int32),
        pltpu.VMEM((K,), jnp.int32),
        pltpu.VMEM((K,), jnp.int32),
        pltpu.VMEM((K,), jnp.int32),
        pltpu.VMEM((K,), jnp.int32),
        pltpu.VMEM((K,), jnp.int32),
        pltpu.VMEM((K,), jnp.float32),
        pltpu.VMEM((K,), jnp.float32),
        pltpu.VMEM((K,), jnp.float32),
        pltpu.VMEM((K,), jnp.float32),
        pltpu.VMEM((K,), jnp.float32),
        pltpu.VMEM((K,), jnp.float32),
        pltpu.VMEM((K,), jnp.float32),
        pltpu.VMEM((K,), jnp.float32),
        pltpu.VMEM((SROWS,), jnp.float32),
        pltpu.VMEM((1, 16), jnp.float32),
        pltpu.VMEM_SHARED((NP,), jnp.float32),
        pltpu.VMEM_SHARED((NP,), jnp.float32),
        pltpu.SemaphoreType.DMA,
        pltpu.SemaphoreType.DMA,
    ],
)(_sc2_body)


B1 = 1000


def _tcA_body(x_ref, w1t_ref, asd_ref, tA_ref, mx_ref):
    h = jnp.dot(x_ref[...], w1t_ref[...], preferred_element_type=jnp.float32)
    asd = jnp.dot(h, asd_ref[...], preferred_element_type=jnp.float32)
    tA_ref[:, 0:F] = h
    tA_ref[:, F:80] = asd
    tA_ref[:, 80:ROWW] = jnp.zeros((B1, ROWW - 80), jnp.float32)
    cur = jnp.broadcast_to(jnp.max(asd, axis=0, keepdims=True), (8, 16))

    @pl.when(pl.program_id(0) == 0)
    def _():
        mx_ref[...] = cur

    @pl.when(pl.program_id(0) != 0)
    def _():
        mx_ref[...] = jnp.maximum(mx_ref[...], cur)


def _tcA(x, w1t, asd):
    return pl.pallas_call(
        _tcA_body,
        grid=(N // B1,),
        in_specs=[
            pl.BlockSpec((B1, IN), lambda i: (i, 0)),
            pl.BlockSpec((IN, F), lambda i: (0, 0)),
            pl.BlockSpec((F, 16), lambda i: (0, 0)),
        ],
        out_specs=[
            pl.BlockSpec((B1, ROWW), lambda i: (i, 0)),
            pl.BlockSpec((8, 16), lambda i: (0, 0)),
        ],
        out_shape=[
            jax.ShapeDtypeStruct((N, ROWW), jnp.float32),
            jax.ShapeDtypeStruct((8, 16), jnp.float32),
        ],
    )(x, w1t, asd)


def _tcB_body(acc_ref, tA_ref, m1_ref, e8_ref, b1_ref,
              w2r_ref, w2s_ref, w2d_ref, t2_ref, mx2_ref):
    a0 = acc_ref[0, :, :]
    a1 = acc_ref[1, :, :]
    hsrc = tA_ref[:, 0:F]
    a_s = tA_ref[:, F:F + HEADS]
    a_d = tA_ref[:, F + HEADS:80]
    del_unused = 0
    z = a_s + a_d
    z = jnp.where(z >= 0.0, z, z * NEG)
    ex8 = jnp.exp(z - m1_ref[...])
    e8 = e8_ref[...]
    num = a0[:, 0:F] + a1[:, 0:F] + hsrc * jnp.dot(
        ex8, e8, preferred_element_type=jnp.float32)
    den8 = a0[:, F:F + HEADS] + a1[:, F:F + HEADS] + ex8
    den = jnp.dot(den8, e8, preferred_element_type=jnp.float32) + 1e-16
    h1 = num / den + b1_ref[...]
    h1 = jnp.where(h1 > 0.0, h1, jnp.exp(h1) - 1.0)
    h2 = jnp.sum(h1 * w2r_ref[...], axis=1, keepdims=True)
    a2s = jnp.sum(h1 * w2s_ref[...], axis=1, keepdims=True)
    a2d = jnp.sum(h1 * w2d_ref[...], axis=1, keepdims=True)
    t2 = jnp.concatenate(
        [a2s, a2d, h2, jnp.zeros((B1, ROWW - 3), jnp.float32)], axis=1)
    t2_ref[...] = t2
    cur = jnp.broadcast_to(
        jnp.max(t2[:, 0:16], axis=0, keepdims=True), (8, 16))

    @pl.when(pl.program_id(0) == 0)
    def _():
        mx2_ref[...] = cur

    @pl.when(pl.program_id(0) != 0)
    def _():
        mx2_ref[...] = jnp.maximum(mx2_ref[...], cur)


def _tcB(acc, tA, m1row, e8, b1row, w2r, w2s, w2d):
    return pl.pallas_call(
        _tcB_body,
        grid=(N // B1,),
        in_specs=[
            pl.BlockSpec((NC, B1, ROWW), lambda i: (0, i, 0)),
            pl.BlockSpec((B1, ROWW), lambda i: (i, 0)),
            pl.BlockSpec((1, HEADS), lambda i: (0, 0)),
            pl.BlockSpec((HEADS, F), lambda i: (0, 0)),
            pl.BlockSpec((1, F), lambda i: (0, 0)),
            pl.BlockSpec((1, F), lambda i: (0, 0)),
            pl.BlockSpec((1, F), lambda i: (0, 0)),
            pl.BlockSpec((1, F), lambda i: (0, 0)),
        ],
        out_specs=[
            pl.BlockSpec((B1, ROWW), lambda i: (i, 0)),
            pl.BlockSpec((8, 16), lambda i: (0, 0)),
        ],
        out_shape=[
            jax.ShapeDtypeStruct((N, ROWW), jnp.float32),
            jax.ShapeDtypeStruct((8, 16), jnp.float32),
        ],
    )(acc, tA, m1row, e8, b1row, w2r, w2s, w2d)


def _tcC_body(acc_ref, t2_ref, m2_ref, out_ref):
    a2s = t2_ref[:, 0:1]
    a2d = t2_ref[:, 1:2]
    h2 = t2_ref[:, 2:3]
    z = a2s + a2d
    z = jnp.where(z >= 0.0, z, z * NEG)
    ex = jnp.exp(z - m2_ref[0:1, 0:1])
    num = acc_ref[:, 0:1] + acc_ref[:, 1:2] + ex * h2
    den = acc_ref[:, 2:3] + acc_ref[:, 3:4] + ex + 1e-16
    out_ref[...] = jnp.broadcast_to(num / den, (B1, 8))


def _tcC(accc, t2, m2row):
    return pl.pallas_call(
        _tcC_body,
        grid=(N // B1,),
        in_specs=[
            pl.BlockSpec((B1, 4), lambda i: (i, 0)),
            pl.BlockSpec((B1, ROWW), lambda i: (i, 0)),
            pl.BlockSpec((1, 16), lambda i: (0, 0)),
        ],
        out_specs=pl.BlockSpec((B1, 8), lambda i: (i, 0)),
        out_shape=jax.ShapeDtypeStruct((N, 8), jnp.float32),
    )(accc, t2, m2row)


def kernel(x, edge_index, W1, att_src1, att_dst1, b1,
           W2, att_src2, att_dst2, b2):
    w1t = W1.T
    asrc = att_src1.reshape(F)
    adst = att_dst1.reshape(F)
    j = jnp.arange(F)[:, None] // HID
    hd = jnp.arange(HEADS)[None, :]
    As = jnp.where(j == hd, asrc[:, None], 0.0)
    Ad = jnp.where(j == hd, adst[:, None], 0.0)
    asd = jnp.concatenate([As, Ad], axis=1)
    e8 = jnp.kron(jnp.eye(HEADS, dtype=jnp.float32),
                  jnp.ones((1, HID), jnp.float32))
    b1row = b1.reshape(1, F)
    w2r = W2.reshape(1, F)
    w2s = w2r * att_src2.reshape(())
    w2d = w2r * att_dst2.reshape(())

    tA, mx = _tcA(x, w1t, asd)
    zmax = mx[0, 0:HEADS] + mx[0, HEADS:16]
    m1 = jnp.where(zmax >= 0.0, zmax, zmax * NEG)
    m1cols = jnp.broadcast_to(m1[:, None], (8, 16)) * jnp.ones((8, 16))
    m1row = m1.reshape(1, HEADS)

    esrc = edge_index[0]
    edst = edge_index[1]
    acc1 = _sc1(esrc, edst, tA, m1cols)

    t2, mx2 = _tcB(acc1, tA, m1row, e8, b1row, w2r, w2s, w2d)
    z2 = mx2[0, 0] + mx2[0, 1]
    m2 = jnp.where(z2 >= 0.0, z2, z2 * NEG)
    m2row = jnp.full((1, 16), m2, jnp.float32)

    a2s1 = t2[:, 0]
    a2d1 = t2[:, 1]
    h2a1 = t2[:, 2]
    acc2n, acc2d = _sc2(esrc, edst, a2s1, a2d1, h2a1, m2row)

    accc = jnp.stack(
        [acc2n[0, 0], acc2n[1, 0], acc2d[0, 0], acc2d[1, 0]], axis=-1)
    out = _tcC(accc[:N], t2, m2row)
    return out[:, 0] + b2[0]
